# baseline (speedup 1.0000x reference)
import jax
import jax.numpy as jnp
import numpy as np
from jax import lax
from jax.experimental import pallas as pl
from jax.experimental.pallas import tpu as pltpu
from jax.experimental.pallas import tpu_sc as plsc

TOPK = 50
NROWS = 128
NCOLS = 32768
LANES = 16
NWORKERS = 32
ROWS_PER_W = NROWS // NWORKERS
CHUNKS = NCOLS // LANES
SGV = 16
NSG = CHUNKS // SGV
CAPV = 120
NUM_CORES = 2
NUM_SUBCORES = 16


def _body(x_hbm, o_hbm, in_v, srt, srtf, bval, champ, tvalb):
    wid = lax.axis_index("s") * NUM_CORES + lax.axis_index("c")
    lane = lax.iota(jnp.int32, LANES)
    onev = lane * 0 + 1
    zerov = lane * 0
    zerof = jnp.zeros((LANES,), jnp.float32)
    onef = zerof + 1.0
    neginf = zerof - jnp.inf

    srt[pl.ds(0, LANES)] = zerov
    srt[pl.ds(32, LANES)] = zerov
    srtf[pl.ds(0, LANES)] = zerof
    srtf[pl.ds(32, LANES)] = zerof

    def scan_incl(x):
        acc = x
        for k in (1, 2, 4, 8):
            srt[pl.ds(16, LANES)] = acc
            acc = acc + srt[pl.ds(16 - k, LANES)]
        return acc

    def total_scalar(x):
        srt[pl.ds(16, LANES)] = scan_incl(x)
        return srt[pl.ds(16, LANES)][15]

    def max_scalar(x):
        acc = x
        for k in (1, 2, 4, 8):
            srtf[pl.ds(16, LANES)] = acc
            sh = srtf[pl.ds(16 - k, LANES)]
            acc = jnp.maximum(acc, jnp.where(lane >= k, sh, neginf))
        srtf[pl.ds(16, LANES)] = acc
        return srtf[pl.ds(16, LANES)][15]

    def min_scalar(x):
        acc = x
        for k in (1, 2, 4, 8):
            srtf[pl.ds(16, LANES)] = acc
            sh = srtf[pl.ds(16 - k, LANES)]
            acc = jnp.minimum(acc, jnp.where(lane >= k, sh, -neginf))
        srtf[pl.ds(16, LANES)] = acc
        return srtf[pl.ds(16, LANES)][15]

    def cur_tval():
        return tvalb[pl.ds(0, LANES)]

    def row_body(rr, _):
        row = wid * ROWS_PER_W + rr
        pltpu.sync_copy(x_hbm.at[row], in_v)
        tvalb[pl.ds(0, LANES)] = neginf

        def rebuild(ncv):
            for g in range(4):
                champ[pl.ds(g * LANES, LANES)] = neginf

            def chbody(j, _c):
                g = lax.rem(j, 4)
                sl = pl.ds(g * LANES, LANES)
                champ[sl] = jnp.maximum(champ[sl], bval[pl.ds(j * LANES,
                                                              LANES)])
                return _c

            lax.fori_loop(0, ncv, chbody, 0)
            cmin = champ[pl.ds(0, LANES)]
            for g in range(1, 4):
                cmin = jnp.minimum(cmin, champ[pl.ds(g * LANES, LANES)])
            tnew = min_scalar(cmin)
            tv = jnp.maximum(cur_tval(), tnew * onef)
            tvalb[pl.ds(0, LANES)] = tv

            def cpbody(j, nkeep):
                vj = bval[pl.ds(j * LANES, LANES)]
                m = vj >= tv
                s = total_scalar(jnp.where(m, onev, zerov))

                def keep(nk):
                    bval[pl.ds(nk * LANES, LANES)] = vj
                    return nk + 1

                return lax.cond(s > 0, keep, lambda nk: nk, nkeep)

            return lax.fori_loop(0, ncv, cpbody, jnp.int32(0))

        def sg_body(sg, ncv):
            base = sg * (SGV * LANES)
            t_vec = cur_tval()
            anym = None
            for q in range(SGV):
                v = in_v[pl.ds(base + q * LANES, LANES)]
                m = v >= t_vec
                anym = m if anym is None else (anym | m)
            s = total_scalar(jnp.where(anym, onev, zerov))

            def hit(op):
                def qbody(q, ncv2):
                    qb = base + q * LANES
                    v = in_v[pl.ds(qb, LANES)]
                    m2 = v >= cur_tval()
                    s2 = total_scalar(jnp.where(m2, onev, zerov))

                    def append(nc3):
                        bval[pl.ds(nc3 * LANES, LANES)] = v
                        nc4 = nc3 + 1
                        return lax.cond(nc4 >= CAPV, rebuild,
                                        lambda n: n, nc4)

                    return lax.cond(s2 > 0, append, lambda n: n, ncv2)

                return lax.fori_loop(0, SGV, qbody, op)

            return lax.cond(s > 0, hit, lambda op: op, ncv)

        ncv = lax.fori_loop(0, NSG, sg_body, jnp.int32(0))
        ncv = rebuild(ncv)

        def count_eq(p_splat):
            def cbody(j, acc):
                vj = bval[pl.ds(j * LANES, LANES)]
                return acc + jnp.where(vj == p_splat, onev, zerov)
            accv = lax.fori_loop(0, ncv, cbody, zerov)
            return total_scalar(accv)

        def ext_round(_i, carry):
            whi, cnt, ng, tstar = carry

            def work(op):
                whi2, cnt2, ng2, _ts = op
                wv = whi2 * onef

                def mbody(j, acc):
                    vj = bval[pl.ds(j * LANES, LANES)]
                    return jnp.maximum(acc, jnp.where(vj < wv, vj, neginf))

                mv = lax.fori_loop(0, ncv, mbody, neginf)
                m = max_scalar(mv)
                ce = count_eq(m * onef)
                return m, cnt2 + ce, cnt2, m

            def skip(op):
                return op

            return lax.cond(cnt < TOPK, work, skip, (whi, cnt, ng, tstar))

        big = jnp.float32(jnp.inf)
        _, cnt_f, ng_f, tstar = lax.fori_loop(
            0, TOPK, ext_round, (big, jnp.int32(0), jnp.int32(0), big))
        r = TOPK - ng_f
        tval_v = tstar * onef

        def mask_sg(sg, bcarry):
            base = sg * (SGV * LANES)
            eqany = None
            for q in range(SGV):
                v = in_v[pl.ds(base + q * LANES, LANES)]
                eq = v == tval_v
                eqany = eq if eqany is None else (eqany | eq)
            se = total_scalar(jnp.where(eqany, onev, zerov))

            def simple(b):
                for q in range(SGV):
                    sl = pl.ds(base + q * LANES, LANES)
                    v = in_v[sl]
                    in_v[sl] = jnp.where(v > tval_v, v, zerof)
                return b

            def with_ties(b):
                def qbody(q, b2):
                    sl = pl.ds(base + q * LANES, LANES)
                    v = in_v[sl]
                    gt = v > tval_v
                    eq = v == tval_v
                    eqc = jnp.where(eq, onev, zerov)
                    excl = scan_incl(eqc) - eqc
                    keep = gt | (eq & ((excl + b2 * onev) < r * onev))
                    in_v[sl] = jnp.where(keep, v, zerof)
                    return b2 + total_scalar(eqc)

                return lax.fori_loop(0, SGV, qbody, b)

            return lax.cond(se > 0, with_ties, simple, bcarry)

        lax.fori_loop(0, NSG, mask_sg, jnp.int32(0))
        pltpu.sync_copy(in_v, o_hbm.at[row])
        return 0

    lax.fori_loop(0, ROWS_PER_W, row_body, 0)


def make_kernel(interpret=False):
    return pl.kernel(
        _body,
        out_type=jax.ShapeDtypeStruct((NROWS, NCOLS), jnp.float32),
        mesh=plsc.VectorSubcoreMesh(
            core_axis_name="c", subcore_axis_name="s",
            num_cores=NUM_CORES, num_subcores=NUM_SUBCORES),
        scratch_types=[
            pltpu.VMEM((NCOLS,), jnp.float32),
            pltpu.VMEM((48,), jnp.int32),
            pltpu.VMEM((48,), jnp.float32),
            pltpu.VMEM((NCOLS,), jnp.float32),
            pltpu.VMEM((4 * LANES,), jnp.float32),
            pltpu.VMEM((LANES,), jnp.float32),
        ],
        interpret=interpret,
    )


_topk_mask_sc = make_kernel()


def kernel(x):
    return _topk_mask_sc(x)


# speedup vs baseline: 121.3925x; 121.3925x over previous
import jax
import jax.numpy as jnp
from jax.experimental import pallas as pl
from jax.experimental.pallas import tpu as pltpu

TOPK = 50
NROWS = 128
NCOLS = 32768
BLOCK_ROWS = 8


def _topk_mask_body(x_ref, o_ref):
    x = x_ref[...]
    u = jax.lax.bitcast_convert_type(x, jnp.uint32)
    sign = u >= jnp.uint32(0x80000000)
    key = jnp.where(sign, ~u, u | jnp.uint32(0x80000000))

    prefix = jnp.zeros((BLOCK_ROWS, 1), jnp.uint32)
    for b in range(31, -1, -1):
        cand = prefix | jnp.uint32(1 << b)
        cnt = jnp.sum((key >= cand).astype(jnp.int32), axis=1, keepdims=True)
        prefix = jnp.where(cnt >= TOPK, cand, prefix)
    kth = prefix

    greater = key > kth
    eq = key == kth
    n_greater = jnp.sum(greater.astype(jnp.int32), axis=1, keepdims=True)
    m = jnp.sum(eq.astype(jnp.int32), axis=1, keepdims=True)
    r = TOPK - n_greater

    col = jax.lax.broadcasted_iota(jnp.int32, (BLOCK_ROWS, NCOLS), 1)

    def tie_search(_):
        v = jnp.where(eq, col, -1)
        want = m - r + 1
        iprefix = jnp.zeros((BLOCK_ROWS, 1), jnp.int32)
        for b in range(14, -1, -1):
            cand = iprefix | jnp.int32(1 << b)
            cnt = jnp.sum((v >= cand).astype(jnp.int32), axis=1,
                          keepdims=True)
            iprefix = jnp.where(cnt >= want, cand, iprefix)
        return jnp.where(r == m, jnp.int32(NCOLS - 1), iprefix)

    def no_tie(_):
        return jnp.full((BLOCK_ROWS, 1), NCOLS - 1, jnp.int32)

    ithresh = jax.lax.cond(jnp.any(r < m), tie_search, no_tie, 0)

    mask = greater | (eq & (col <= ithresh))
    o_ref[...] = jnp.where(mask, x, 0.0)


def kernel(x):
    return pl.pallas_call(
        _topk_mask_body,
        grid=(NROWS // BLOCK_ROWS,),
        in_specs=[pl.BlockSpec((BLOCK_ROWS, NCOLS), lambda i: (i, 0))],
        out_specs=pl.BlockSpec((BLOCK_ROWS, NCOLS), lambda i: (i, 0)),
        out_shape=jax.ShapeDtypeStruct((NROWS, NCOLS), jnp.float32),
    )(x)


# speedup vs baseline: 227.0985x; 1.8708x over previous
import jax
import jax.numpy as jnp
from jax.experimental import pallas as pl
from jax.experimental.pallas import tpu as pltpu

TOPK = 50
NROWS = 128
NCOLS = 32768
BLOCK_ROWS = 32


def _topk_mask_body(x_ref, o_ref):
    x = x_ref[...]
    u = jax.lax.bitcast_convert_type(x, jnp.uint32)
    sign = u >= jnp.uint32(0x80000000)
    key = jnp.where(sign, ~u, u | jnp.uint32(0x80000000))

    prefix = jnp.zeros((BLOCK_ROWS, 1), jnp.uint32)
    for b in range(31, -1, -1):
        cand = prefix | jnp.uint32(1 << b)
        cnt = jnp.sum((key >= cand).astype(jnp.int32), axis=1, keepdims=True)
        prefix = jnp.where(cnt >= TOPK, cand, prefix)
    kth = prefix

    greater = key > kth
    eq = key == kth
    n_greater = jnp.sum(greater.astype(jnp.int32), axis=1, keepdims=True)
    m = jnp.sum(eq.astype(jnp.int32), axis=1, keepdims=True)
    r = TOPK - n_greater

    col = jax.lax.broadcasted_iota(jnp.int32, (BLOCK_ROWS, NCOLS), 1)

    def tie_search(_):
        v = jnp.where(eq, col, -1)
        want = m - r + 1
        iprefix = jnp.zeros((BLOCK_ROWS, 1), jnp.int32)
        for b in range(14, -1, -1):
            cand = iprefix | jnp.int32(1 << b)
            cnt = jnp.sum((v >= cand).astype(jnp.int32), axis=1,
                          keepdims=True)
            iprefix = jnp.where(cnt >= want, cand, iprefix)
        return jnp.where(r == m, jnp.int32(NCOLS - 1), iprefix)

    def no_tie(_):
        return jnp.full((BLOCK_ROWS, 1), NCOLS - 1, jnp.int32)

    ithresh = jax.lax.cond(jnp.any(r < m), tie_search, no_tie, 0)

    mask = greater | (eq & (col <= ithresh))
    o_ref[...] = jnp.where(mask, x, 0.0)


def kernel(x):
    return pl.pallas_call(
        _topk_mask_body,
        grid=(NROWS // BLOCK_ROWS,),
        in_specs=[pl.BlockSpec((BLOCK_ROWS, NCOLS), lambda i: (i, 0))],
        out_specs=pl.BlockSpec((BLOCK_ROWS, NCOLS), lambda i: (i, 0)),
        out_shape=jax.ShapeDtypeStruct((NROWS, NCOLS), jnp.float32),
    )(x)
